# baseline (speedup 1.0000x reference)
import functools

import jax
import jax.numpy as jnp
from jax import lax
from jax.experimental import pallas as pl
from jax.experimental.pallas import tpu as pltpu
from jax.experimental.pallas import tpu_sc as plsc

_B = 1024
_K = 16
_M = 112
_NCAND = 128
_D = 64 + _NCAND * _NCAND
_ROWBLK = 8
_NW = 32
_BPW = _B // _NW
_NEG = -1e30


def _tc_body(x_ref, c_ref, out_ref):
    lane = lax.broadcasted_iota(jnp.int32, (_ROWBLK, _NCAND), 1)
    lane_lt64 = lane < 64

    cmask = jnp.zeros((_ROWBLK, _NCAND), jnp.float32)
    one = jnp.ones((_ROWBLK, _NCAND), jnp.float32)
    for k in range(_K):
        cmask = jnp.where(lane == c_ref[:, k : k + 1], one, cmask)

    zero_col = jnp.zeros((_ROWBLK, 1), jnp.float32)
    acc = jnp.full((_ROWBLK, _NCAND), _NEG, jnp.float32)
    for a in range(_NCAND):
        sl = x_ref[:, pl.ds(a * _NCAND, _NCAND)]
        rolled = jnp.roll(sl, 64, axis=1)
        cm_hi = cmask[:, a : a + 1]
        cm_lo = cmask[:, a - 1 : a] if a > 0 else zero_col
        sel = jnp.where(lane_lt64, cm_hi, cm_lo)
        acc = jnp.maximum(acc, jnp.where(sel > 0.5, rolled, _NEG))
    t64 = x_ref[:, pl.ds(_NCAND * _NCAND, 64)]
    tail = jnp.concatenate([jnp.full((_ROWBLK, 64), _NEG, jnp.float32), t64],
                           axis=1)
    acc = jnp.maximum(
        acc, jnp.where(cmask[:, _NCAND - 1 : _NCAND] > 0.5, tail, _NEG))
    out_ref[...] = acc


_tc_colmax = pl.pallas_call(
    _tc_body,
    grid=(_B // _ROWBLK,),
    in_specs=[
        pl.BlockSpec((_ROWBLK, _D), lambda a: (a, 0)),
        pl.BlockSpec((_ROWBLK, _K), lambda a: (a, 0)),
    ],
    out_specs=pl.BlockSpec((_ROWBLK, _NCAND), lambda a: (a, 0)),
    out_shape=jax.ShapeDtypeStruct((_B, _NCAND), jnp.float32),
    compiler_params=pltpu.CompilerParams(dimension_semantics=("arbitrary",)),
)


def _sc_body(acc_hbm, d_hbm, thr_hbm, out_hbm, acc_v, d_v, thr_v, stage):
    cid = lax.axis_index("c")
    sid = lax.axis_index("s")
    wid = sid * 2 + cid
    base = wid * _BPW

    pltpu.sync_copy(acc_hbm.at[pl.ds(base, _BPW)], acc_v)
    pltpu.sync_copy(d_hbm.at[pl.ds(pl.multiple_of(base * _M, 8), _BPW * _M)], d_v)
    pltpu.sync_copy(thr_hbm, thr_v)

    lanes = lax.iota(jnp.int32, 16)
    thr = thr_v[...]

    def row_body(i, acc):
        isplat = jnp.full((16,), i, jnp.int32)
        mvec = jnp.full((16,), _NEG, jnp.float32)
        for j in range(_M // 16):
            dj = d_v[pl.ds(pl.multiple_of(i * _M + j * 16, 8), 16)]
            g = plsc.load_gather(acc_v, [isplat, dj])
            mvec = jnp.maximum(mvec, g)
        rowmax = plsc.cummax(mvec)
        sig = 1.0 / (1.0 + jnp.exp((thr - rowmax) * 0.1))
        return acc + jnp.where(lanes == 15, sig, 0.0)

    acc = lax.fori_loop(0, _BPW, row_body, jnp.zeros((16,), jnp.float32))
    stage[...] = acc
    pltpu.sync_copy(stage, out_hbm.at[wid])


@functools.cache
def _sc_gather_sigmoid():
    return functools.partial(
        pl.kernel,
        out_type=jax.ShapeDtypeStruct((_NW, 16), jnp.float32),
        mesh=plsc.VectorSubcoreMesh(core_axis_name="c", subcore_axis_name="s"),
        compiler_params=pltpu.CompilerParams(
            needs_layout_passes=False, use_tc_tiling_on_sc=False),
        scratch_types=[
            pltpu.VMEM((_BPW, _NCAND), jnp.float32),
            pltpu.VMEM((_BPW * _M,), jnp.int32),
            pltpu.VMEM((16,), jnp.float32),
            pltpu.VMEM((16,), jnp.float32),
        ],
    )(_sc_body)


def kernel(c_indices, d_indices, input, n_voters, num_winners, batch_size,
           num_candidates):
    c32 = c_indices.astype(jnp.int32)
    d_flat = jnp.reshape(d_indices, (-1,)).astype(jnp.int32)
    thr = jnp.full((16,), (n_voters // 2 + 1), dtype=jnp.float32)
    acc = _tc_colmax(input, c32)
    partials = _sc_gather_sigmoid()(acc, d_flat, thr)
    return jnp.sum(partials)


# speedup vs baseline: 1.5865x; 1.5865x over previous
import functools

import jax
import jax.numpy as jnp
from jax import lax
from jax.experimental import pallas as pl
from jax.experimental.pallas import tpu as pltpu
from jax.experimental.pallas import tpu_sc as plsc

_B = 1024
_K = 16
_M = 112
_NCAND = 128
_D = 64 + _NCAND * _NCAND
_ROWBLK = 64
_NW = 32
_BPW = _B // _NW
_NEG = -1e30


def _tc_body(x_ref, c_ref, out_ref):
    lane = lax.broadcasted_iota(jnp.int32, (_ROWBLK, _NCAND), 1)
    lane_lt64 = lane < 64

    neg = jnp.full((_ROWBLK, _NCAND), _NEG, jnp.float32)
    pos = jnp.full((_ROWBLK, _NCAND), -_NEG, jnp.float32)
    bi = neg
    for k in range(_K):
        bi = jnp.where(lane == c_ref[:, k : k + 1], pos, bi)

    acc_hi = neg
    acc_lo = neg
    prev_b = neg
    for a in range(_NCAND):
        sl = x_ref[:, pl.ds(a * _NCAND, _NCAND)]
        b_a = jnp.broadcast_to(bi[:, a : a + 1], (_ROWBLK, _NCAND))
        acc_hi = jnp.maximum(acc_hi, jnp.minimum(sl, b_a))
        acc_lo = jnp.maximum(acc_lo, jnp.minimum(sl, prev_b))
        prev_b = b_a
    acc = jnp.where(lane_lt64, jnp.roll(acc_hi, 64, axis=1),
                    jnp.roll(acc_lo, 64, axis=1))
    t64 = x_ref[:, pl.ds(_NCAND * _NCAND, 64)]
    tail = jnp.concatenate([neg[:, :64], t64], axis=1)
    acc = jnp.maximum(
        acc,
        jnp.minimum(tail, jnp.broadcast_to(bi[:, _NCAND - 1 : _NCAND],
                                           (_ROWBLK, _NCAND))))
    out_ref[...] = acc


_tc_colmax = pl.pallas_call(
    _tc_body,
    grid=(_B // _ROWBLK,),
    in_specs=[
        pl.BlockSpec((_ROWBLK, _D), lambda a: (a, 0)),
        pl.BlockSpec((_ROWBLK, _K), lambda a: (a, 0)),
    ],
    out_specs=pl.BlockSpec((_ROWBLK, _NCAND), lambda a: (a, 0)),
    out_shape=jax.ShapeDtypeStruct((_B, _NCAND), jnp.float32),
    compiler_params=pltpu.CompilerParams(dimension_semantics=("arbitrary",)),
)


def _sc_body(acc_hbm, d_hbm, thr_hbm, out_hbm, acc_v, d_v, thr_v, stage):
    cid = lax.axis_index("c")
    sid = lax.axis_index("s")
    wid = sid * 2 + cid
    base = wid * _BPW

    pltpu.sync_copy(acc_hbm.at[pl.ds(base, _BPW)], acc_v)
    pltpu.sync_copy(d_hbm.at[pl.ds(pl.multiple_of(base * _M, 8), _BPW * _M)], d_v)
    pltpu.sync_copy(thr_hbm, thr_v)

    lanes = lax.iota(jnp.int32, 16)
    thr = thr_v[...]

    def row_body(i, acc):
        isplat = jnp.full((16,), i, jnp.int32)
        mvec = jnp.full((16,), _NEG, jnp.float32)
        for j in range(_M // 16):
            dj = d_v[pl.ds(pl.multiple_of(i * _M + j * 16, 8), 16)]
            g = plsc.load_gather(acc_v, [isplat, dj])
            mvec = jnp.maximum(mvec, g)
        rowmax = plsc.cummax(mvec)
        sig = 1.0 / (1.0 + jnp.exp((thr - rowmax) * 0.1))
        return acc + jnp.where(lanes == 15, sig, 0.0)

    acc = lax.fori_loop(0, _BPW, row_body, jnp.zeros((16,), jnp.float32))
    stage[...] = acc
    pltpu.sync_copy(stage, out_hbm.at[wid])


@functools.cache
def _sc_gather_sigmoid():
    return functools.partial(
        pl.kernel,
        out_type=jax.ShapeDtypeStruct((_NW, 16), jnp.float32),
        mesh=plsc.VectorSubcoreMesh(core_axis_name="c", subcore_axis_name="s"),
        compiler_params=pltpu.CompilerParams(
            needs_layout_passes=False, use_tc_tiling_on_sc=False),
        scratch_types=[
            pltpu.VMEM((_BPW, _NCAND), jnp.float32),
            pltpu.VMEM((_BPW * _M,), jnp.int32),
            pltpu.VMEM((16,), jnp.float32),
            pltpu.VMEM((16,), jnp.float32),
        ],
    )(_sc_body)


def kernel(c_indices, d_indices, input, n_voters, num_winners, batch_size,
           num_candidates):
    c32 = c_indices.astype(jnp.int32)
    d_flat = jnp.reshape(d_indices, (-1,)).astype(jnp.int32)
    thr = jnp.full((16,), (n_voters // 2 + 1), dtype=jnp.float32)
    acc = _tc_colmax(input, c32)
    partials = _sc_gather_sigmoid()(acc, d_flat, thr)
    return jnp.sum(partials)


# speedup vs baseline: 1.8184x; 1.1462x over previous
import functools

import jax
import jax.numpy as jnp
from jax import lax
from jax.experimental import pallas as pl
from jax.experimental.pallas import tpu as pltpu
from jax.experimental.pallas import tpu_sc as plsc

_B = 1024
_K = 16
_M = 112
_NCAND = 128
_D = 64 + _NCAND * _NCAND
_NW = 32
_BPW = _B // _NW
_WIN = 2 * _NCAND
_ROWBUF = _K * _WIN
_NEG = -1e30


def _sc_body(x_hbm, c_hbm, d_hbm, thr_hbm, out_hbm,
             c_v, d_v, thr_v, buf0, buf1, stage, sem0, sem1):
    cid = lax.axis_index("c")
    sid = lax.axis_index("s")
    wid = sid * 2 + cid
    base = wid * _BPW

    pltpu.sync_copy(c_hbm.at[pl.ds(pl.multiple_of(base * _K, 8), _BPW * _K)], c_v)
    pltpu.sync_copy(d_hbm.at[pl.ds(pl.multiple_of(base * _M, 8), _BPW * _M)], d_v)
    pltpu.sync_copy(thr_hbm, thr_v)

    def issue(i, buf, sem):
        cvec = c_v[pl.ds(pl.multiple_of(i * _K, 8), _K)]
        gi = base + i
        for k in range(_K):
            cc = cvec[k] * _NCAND
            pltpu.async_copy(x_hbm.at[gi, pl.ds(cc, _WIN)],
                             buf.at[pl.ds(k * _WIN, _WIN)], sem)

    def drain(buf, sem):
        pltpu.make_async_copy(x_hbm.at[0, pl.ds(0, _ROWBUF)],
                              buf.at[pl.ds(0, _ROWBUF)], sem).wait()

    lanes = lax.iota(jnp.int32, 16)
    thr = thr_v[...]

    def compute(i, buf, acc):
        mvec = jnp.full((16,), _NEG, jnp.float32)
        for j in range(_M // 16):
            dj = d_v[pl.ds(pl.multiple_of(i * _M + j * 16, 8), 16)]
            for k in range(_K):
                g = plsc.load_gather(buf, [dj + (k * _WIN + 64)])
                mvec = jnp.maximum(mvec, g)
        rowmax = plsc.cummax(mvec)
        sig = 1.0 / (1.0 + jnp.exp((thr - rowmax) * 0.1))
        return acc + jnp.where(lanes == 15, sig, 0.0)

    issue(0, buf0, sem0)

    def pair(t, acc):
        i0 = t * 2
        issue(i0 + 1, buf1, sem1)
        drain(buf0, sem0)
        acc = compute(i0, buf0, acc)

        @pl.when(i0 + 2 < _BPW)
        def _():
            issue(i0 + 2, buf0, sem0)

        drain(buf1, sem1)
        return compute(i0 + 1, buf1, acc)

    acc = lax.fori_loop(0, _BPW // 2, pair, jnp.zeros((16,), jnp.float32))
    stage[...] = acc
    pltpu.sync_copy(stage, out_hbm.at[pl.ds(pl.multiple_of(wid * 16, 8), 16)])


@functools.cache
def _sc_loss_kernel():
    return functools.partial(
        pl.kernel,
        out_type=jax.ShapeDtypeStruct((_NW * 16,), jnp.float32),
        mesh=plsc.VectorSubcoreMesh(core_axis_name="c", subcore_axis_name="s"),
        compiler_params=pltpu.CompilerParams(
            needs_layout_passes=False, use_tc_tiling_on_sc=True),
        scratch_types=[
            pltpu.VMEM((_BPW * _K,), jnp.int32),
            pltpu.VMEM((_BPW * _M,), jnp.int32),
            pltpu.VMEM((16,), jnp.float32),
            pltpu.VMEM((_ROWBUF,), jnp.float32),
            pltpu.VMEM((_ROWBUF,), jnp.float32),
            pltpu.VMEM((16,), jnp.float32),
            pltpu.SemaphoreType.DMA,
            pltpu.SemaphoreType.DMA,
        ],
    )(_sc_body)


def kernel(c_indices, d_indices, input, n_voters, num_winners, batch_size,
           num_candidates):
    c_flat = jnp.reshape(c_indices, (-1,)).astype(jnp.int32)
    d_flat = jnp.reshape(d_indices, (-1,)).astype(jnp.int32)
    thr = jnp.full((16,), (n_voters // 2 + 1), dtype=jnp.float32)
    partials = _sc_loss_kernel()(input, c_flat, d_flat, thr)
    return jnp.sum(partials)
